# baseline (speedup 1.0000x reference)
import functools

import jax
import jax.numpy as jnp
from jax import lax
from jax.experimental import pallas as pl
from jax.experimental.pallas import tpu as pltpu
from jax.experimental.pallas import tpu_sc as plsc

BATCH = 16384
DIM = 64
LANES = 16
NUM_CORES = 2
NUM_SUBCORES = 16
NUM_WORKERS = NUM_CORES * NUM_SUBCORES
B_PER_W = BATCH // NUM_WORKERS
CHUNK = 16
N_CHUNKS = B_PER_W // CHUNK
N_BUF = 2


def _body(uids_hbm, iids_hbm, ut_hbm, it_hbm, out_hbm,
          ids_u, ids_v, dimidx, ubuf, vbuf, out_v, sems):
    w = lax.axis_index("s") * NUM_CORES + lax.axis_index("c")
    base = w * B_PER_W

    pltpu.sync_copy(uids_hbm.at[pl.ds(base, B_PER_W)], ids_u)
    pltpu.sync_copy(iids_hbm.at[pl.ds(base, B_PER_W)], ids_v)

    lane = lax.iota(jnp.int32, LANES)
    for k in range(DIM // LANES):
        dimidx[pl.ds(k * LANES, LANES)] = k * LANES + lane

    def fire(c, b):
        idu = jnp.bitwise_and(ids_u[pl.ds(c * CHUNK, CHUNK)], ~(LANES - 1))
        idv = jnp.bitwise_and(ids_v[pl.ds(c * CHUNK, CHUNK)], ~(LANES - 1))
        handles = []
        for l in range(CHUNK):
            ru = pl.multiple_of(idu[l], LANES)
            rv = pl.multiple_of(idv[l], LANES)
            handles.append(pltpu.async_copy(
                ut_hbm.at[:, pl.ds(ru, LANES)],
                ubuf.at[b, l], sems.at[b]))
            handles.append(pltpu.async_copy(
                it_hbm.at[:, pl.ds(rv, LANES)],
                vbuf.at[b, l], sems.at[b]))
        return handles

    def drain(b):
        pltpu.make_async_copy(ut_hbm.at[:, pl.ds(0, CHUNK)],
                              ubuf.at[b], sems.at[b]).wait()
        pltpu.make_async_copy(it_hbm.at[:, pl.ds(0, CHUNK)],
                              vbuf.at[b], sems.at[b]).wait()

    def compute(c, b):
        su = jnp.bitwise_and(ids_u[pl.ds(c * CHUNK, CHUNK)], LANES - 1)
        sv = jnp.bitwise_and(ids_v[pl.ds(c * CHUNK, CHUNK)], LANES - 1)
        acc = jnp.zeros((LANES,), jnp.float32)
        for d in range(DIM):
            dv = jnp.full((LANES,), d, jnp.int32)
            u = plsc.load_gather(ubuf.at[b], [lane, dv, su])
            v = plsc.load_gather(vbuf.at[b], [lane, dv, sv])
            acc = acc + u * v
        out_v[pl.ds(c * CHUNK, CHUNK)] = acc

    def step(g, _):
        for h in fire(g, 0):
            h.wait()
        compute(g, 0)
        return 0

    lax.fori_loop(0, N_CHUNKS, step, 0)

    pltpu.sync_copy(out_v, out_hbm.at[pl.ds(base, B_PER_W)])


def kernel(user_ids, item_ids, user_table, item_table):
    ut = user_table.T
    it = item_table.T
    uids = user_ids.astype(jnp.int32)
    iids = item_ids.astype(jnp.int32)

    mesh = plsc.VectorSubcoreMesh(
        core_axis_name="c", subcore_axis_name="s",
        num_cores=NUM_CORES, num_subcores=NUM_SUBCORES)

    run = pl.kernel(
        _body,
        out_type=jax.ShapeDtypeStruct((BATCH,), jnp.float32),
        mesh=mesh,
        scratch_types=[
            pltpu.VMEM((B_PER_W,), jnp.int32),
            pltpu.VMEM((B_PER_W,), jnp.int32),
            pltpu.VMEM((DIM,), jnp.int32),
            pltpu.VMEM((N_BUF, CHUNK, DIM, LANES), jnp.float32),
            pltpu.VMEM((N_BUF, CHUNK, DIM, LANES), jnp.float32),
            pltpu.VMEM((B_PER_W,), jnp.float32),
            pltpu.SemaphoreType.DMA((N_BUF,)),
        ],
        compiler_params=pltpu.CompilerParams(
            needs_layout_passes=False, use_tc_tiling_on_sc=False),
    )
    return run(uids, iids, ut, it)
